# speedup vs baseline: 2.2377x; 2.2377x over previous
from functools import partial

import jax
import jax.numpy as jnp
from jax.experimental import pallas as pl
from jax.experimental.pallas import tpu as pltpu
from jax.experimental.pallas import tpu_sc as plsc


def _scale_body(t_ref, o_ref, *, scale):
    o_ref[...] = t_ref[...] * scale


def _scale_table(table, scale):
    vocab, d = table.shape
    block = 1000
    return pl.pallas_call(
        partial(_scale_body, scale=scale),
        out_shape=jax.ShapeDtypeStruct(table.shape, table.dtype),
        grid=(vocab // block,),
        in_specs=[pl.BlockSpec((block, d), lambda i: (i, 0))],
        out_specs=pl.BlockSpec((block, d), lambda i: (i, 0)),
    )(table)


def kernel(x, table):
    b, s = x.shape
    vocab, d = table.shape
    n = b * s
    scale = float(d) ** 0.5

    scaled = _scale_table(table, scale)
    idx = x.reshape(1, n)

    window = 128
    mesh = plsc.VectorSubcoreMesh(core_axis_name="core",
                                  subcore_axis_name="subcore")

    @pl.kernel(out_type=jax.ShapeDtypeStruct((n, d), table.dtype), mesh=mesh)
    def gather_kernel(t_hbm, i_hbm, o_hbm):
        def body(i_vmem, o_vmem):
            pltpu.sync_copy(t_hbm.at[i_vmem.at[0]], o_vmem)

        pltpu.emit_pipeline(
            body,
            grid=(n // window,),
            in_specs=[pl.BlockSpec((1, window), index_map=lambda i: (0, i))],
            out_specs=[pl.BlockSpec((window, d), index_map=lambda i: (i, 0))],
            core_axis_name=("core", "subcore"),
            dimension_semantics=(pltpu.PARALLEL,),
        )(i_hbm, o_hbm)

    out = gather_kernel(scaled, idx)
    return out.reshape(b, s, d)


# speedup vs baseline: 2.2995x; 1.0276x over previous
from functools import partial

import jax
import jax.numpy as jnp
from jax.experimental import pallas as pl
from jax.experimental.pallas import tpu as pltpu
from jax.experimental.pallas import tpu_sc as plsc


def _scale_body(t_ref, o_ref, *, scale):
    o_ref[...] = t_ref[...] * scale


def _scale_reshape(y, b, s, d, scale):
    y3 = y.reshape(b, s, d)
    blk = 64
    return pl.pallas_call(
        partial(_scale_body, scale=scale),
        out_shape=jax.ShapeDtypeStruct((b, s, d), y.dtype),
        grid=(b // blk,),
        in_specs=[pl.BlockSpec((blk, s, d), lambda i: (i, 0, 0))],
        out_specs=pl.BlockSpec((blk, s, d), lambda i: (i, 0, 0)),
    )(y3)


def kernel(x, table):
    b, s = x.shape
    vocab, d = table.shape
    n = b * s
    scale = float(d) ** 0.5

    idx = x.reshape(1, n)

    window = 128
    mesh = plsc.VectorSubcoreMesh(core_axis_name="core",
                                  subcore_axis_name="subcore")

    @pl.kernel(out_type=jax.ShapeDtypeStruct((n, d), table.dtype), mesh=mesh)
    def gather_kernel(t_hbm, i_hbm, o_hbm):
        def body(i_vmem, o_vmem):
            pltpu.sync_copy(t_hbm.at[i_vmem.at[0]], o_vmem)

        pltpu.emit_pipeline(
            body,
            grid=(n // window,),
            in_specs=[pl.BlockSpec((1, window), index_map=lambda i: (0, i))],
            out_specs=[pl.BlockSpec((window, d), index_map=lambda i: (i, 0))],
            core_axis_name=("core", "subcore"),
            dimension_semantics=(pltpu.PARALLEL,),
        )(i_hbm, o_hbm)

    y = gather_kernel(table, idx)
    return _scale_reshape(y, b, s, d, scale)


# speedup vs baseline: 5.2852x; 2.2984x over previous
import functools

import jax
import jax.numpy as jnp
from jax import lax
from jax.experimental import pallas as pl
from jax.experimental.pallas import tpu as pltpu
from jax.experimental.pallas import tpu_sc as plsc

NC, NS = 2, 16
NW = NC * NS
CB = 2
NB = 4


def kernel(x, table):
    b, s = x.shape
    vocab, d = table.shape
    n = b * s
    scale = float(d) ** 0.5
    bpw = b // NW
    ci = CB * s
    nch = bpw // CB

    x2 = x.reshape(n // ci, ci)
    mesh = plsc.VectorSubcoreMesh(core_axis_name="c", subcore_axis_name="s")

    @functools.partial(
        pl.kernel, mesh=mesh,
        out_type=jax.ShapeDtypeStruct((b, s, d), table.dtype),
        scratch_types=(
            [pltpu.VMEM((nch, ci), jnp.int32)]
            + [pltpu.VMEM((ci, d), jnp.float32) for _ in range(2 * NB)]
            + [pltpu.SemaphoreType.DMA for _ in range(2 * NB + 1)]
        ),
    )
    def emb_kernel(t_hbm, xf_hbm, o_hbm, idx_v, *bufs_and_sems):
        gbuf = bufs_and_sems[:NB]
        obuf = bufs_and_sems[NB:2 * NB]
        gsem = bufs_and_sems[2 * NB:3 * NB]
        osem = bufs_and_sems[3 * NB:4 * NB]
        isem = bufs_and_sems[4 * NB]

        wid = lax.axis_index("s") * NC + lax.axis_index("c")
        row0 = wid * bpw

        pltpu.async_copy(xf_hbm.at[pl.ds(wid * nch, nch)], idx_v,
                         isem).wait()

        def issue_gather(j, c):
            pltpu.async_copy(t_hbm.at[idx_v.at[c]], gbuf[j], gsem[j])

        def wait_gather(j, c):
            pltpu.make_async_copy(t_hbm.at[idx_v.at[c]], gbuf[j],
                                  gsem[j]).wait()

        def issue_out(j, c):
            b0 = row0 + c * CB
            for k in range(CB):
                pltpu.async_copy(obuf[j].at[pl.ds(k * s, s)],
                                 o_hbm.at[b0 + k], osem[j])

        def wait_out(j, c):
            b0 = row0 + c * CB
            for k in range(CB):
                pltpu.make_async_copy(obuf[j].at[pl.ds(k * s, s)],
                                      o_hbm.at[b0 + k], osem[j]).wait()

        for j in range(NB):
            issue_gather(j, j)

        @pl.loop(0, nch, step=NB)
        def _(c0):
            for j in range(NB):
                c = c0 + j
                wait_gather(j, c)

                @pl.when(c >= NB)
                def _():
                    wait_out(j, c - NB)

                @pl.loop(0, ci)
                def _(r):
                    for cc in range(0, d, 16):
                        obuf[j][r, pl.ds(cc, 16)] = (
                            gbuf[j][r, pl.ds(cc, 16)] * scale)

                @pl.when(c + NB < nch)
                def _():
                    issue_gather(j, c + NB)

                issue_out(j, c)

        for j in range(NB):
            wait_out(j, nch - NB + j)

    return emb_kernel(table, x2)
